# baseline (speedup 1.0000x reference)
import functools

import jax
import jax.numpy as jnp
from jax import lax
from jax.experimental import pallas as pl
from jax.experimental.pallas import tpu as pltpu
from jax.experimental.pallas import tpu_sc as plsc

_B = 16384
_D = 64
_V = 1000000
_NC = 2
_NS = 16
_NW = _NC * _NS
_RPT = _B // _NW
_CH = 128
_NCHUNK = _RPT // _CH

_R2 = _V // 2
_NBLK = _V // 128

_mesh = plsc.VectorSubcoreMesh(core_axis_name="c", subcore_axis_name="s")

_params = pltpu.CompilerParams(
    needs_layout_passes=False, use_tc_tiling_on_sc=True
)


@functools.partial(
    pl.kernel,
    mesh=_mesh,
    compiler_params=_params,
    out_type=(
        jax.ShapeDtypeStruct((_R2, 128), jnp.float32),
        jax.ShapeDtypeStruct((_R2, 128), jnp.float32),
    ),
    scratch_types=[
        pltpu.VMEM((2, _D, 128), jnp.float32),
        pltpu.VMEM((2, _D, 128), jnp.float32),
        pltpu.VMEM((2, _D, 128), jnp.float32),
        pltpu.VMEM((2, _D, 128), jnp.float32),
        pltpu.VMEM((32, 128), jnp.float32),
        pltpu.SemaphoreType.DMA((2,)),
        pltpu.SemaphoreType.DMA((2,)),
    ],
)
def _sc_relayout(ut_hbm, it_hbm, tail_u_hbm, tail_i_hbm, out_u, out_i,
                 blk_u, blk_i, ob_u, ob_i, tail_v, sem_in, sem_out):
    wid = lax.axis_index("s") * _NC + lax.axis_index("c")
    cnt = 244 + (wid < _NBLK - 244 * _NW).astype(jnp.int32)

    def fire_in(k):
        par = k & 1
        col = (wid + k * _NW) * 128
        pltpu.async_copy(ut_hbm.at[:, pl.ds(col, 128)], blk_u.at[par],
                         sem_in.at[par])
        pltpu.async_copy(it_hbm.at[:, pl.ds(col, 128)], blk_i.at[par],
                         sem_in.at[par])

    def wait_in(par):
        s = sem_in.at[par]
        pltpu.make_async_copy(ut_hbm.at[:, pl.ds(0, 128)], blk_u.at[par], s).wait()
        pltpu.make_async_copy(it_hbm.at[:, pl.ds(0, 128)], blk_i.at[par], s).wait()

    dvec = lax.iota(jnp.int32, 16)

    def process(m):
        par = m & 1
        orow = (wid + m * _NW) * 64

        @pl.when(m >= 2)
        def _():
            s = sem_out.at[par]
            pltpu.make_async_copy(ob_u.at[par], out_u.at[pl.ds(0, 64), :], s).wait()
            pltpu.make_async_copy(ob_i.at[par], out_i.at[pl.ds(0, 64), :], s).wait()

        for blk, ob in ((blk_u, ob_u), (blk_i, ob_i)):
            for t in range(64):
                for q in range(8):
                    lv = jnp.full((16,), 2 * t + q // 4, jnp.int32)
                    v = plsc.load_gather(blk.at[par], [dvec + (q % 4) * 16, lv])
                    ob[par, t, pl.ds(q * 16, 16)] = v

        pltpu.async_copy(ob_u.at[par], out_u.at[pl.ds(orow, 64), :],
                         sem_out.at[par])
        pltpu.async_copy(ob_i.at[par], out_i.at[pl.ds(orow, 64), :],
                         sem_out.at[par])

    fire_in(0)

    def body(k, carry):
        fire_in(k)
        wait_in(1 - (k & 1))
        process(k - 1)
        return carry

    lax.fori_loop(1, cnt, body, 0)
    last = cnt - 1
    wait_in(last & 1)
    process(last)

    for par in (0, 1):
        s = sem_out.at[par]
        pltpu.make_async_copy(ob_u.at[par], out_u.at[pl.ds(0, 64), :], s).wait()
        pltpu.make_async_copy(ob_i.at[par], out_i.at[pl.ds(0, 64), :], s).wait()

    @pl.when(wid == 0)
    def _():
        pltpu.sync_copy(tail_u_hbm, tail_v)
        pltpu.sync_copy(tail_v, out_u.at[pl.ds(_R2 - 32, 32), :])
        pltpu.sync_copy(tail_i_hbm, tail_v)
        pltpu.sync_copy(tail_v, out_i.at[pl.ds(_R2 - 32, 32), :])


@functools.partial(
    pl.kernel,
    mesh=_mesh,
    compiler_params=_params,
    out_type=jax.ShapeDtypeStruct((_B,), jnp.float32),
    scratch_types=[
        pltpu.VMEM((_RPT,), jnp.int32),
        pltpu.VMEM((_RPT,), jnp.int32),
        pltpu.VMEM((_RPT,), jnp.int32),
        pltpu.VMEM((2, _CH), jnp.int32),
        pltpu.VMEM((2, _CH), jnp.int32),
        pltpu.VMEM((2, _CH), jnp.int32),
        pltpu.VMEM((2, _CH, 128), jnp.float32),
        pltpu.VMEM((2, _CH, 128), jnp.float32),
        pltpu.VMEM((2, _CH, 128), jnp.float32),
        pltpu.VMEM((_RPT,), jnp.float32),
        pltpu.SemaphoreType.DMA((2,)),
    ],
)
def _sc_scores(uid_hbm, pid_hbm, nid_hbm, utab_hbm, itab_hbm, out_hbm,
               idx_u, idx_p, idx_n, six_u, six_p, six_n,
               dat_u, dat_p, dat_n, scores, sem):
    wid = lax.axis_index("s") * _NC + lax.axis_index("c")
    base = wid * _RPT

    pltpu.sync_copy(uid_hbm.at[pl.ds(base, _RPT)], idx_u)
    pltpu.sync_copy(pid_hbm.at[pl.ds(base, _RPT)], idx_p)
    pltpu.sync_copy(nid_hbm.at[pl.ds(base, _RPT)], idx_n)

    def fire(c, par):
        for ids, six, tab, dat in (
            (idx_u, six_u, utab_hbm, dat_u),
            (idx_p, six_p, itab_hbm, dat_p),
            (idx_n, six_n, itab_hbm, dat_n),
        ):
            for k in range(_CH // 16):
                v = ids[pl.ds(c * _CH + k * 16, 16)]
                six[par, pl.ds(k * 16, 16)] = v >> 1
            pltpu.async_copy(tab.at[six.at[par]], dat.at[par], sem.at[par])

    def drain(par):
        s = sem.at[par]
        pltpu.make_async_copy(utab_hbm.at[six_u.at[par]], dat_u.at[par], s).wait()
        pltpu.make_async_copy(itab_hbm.at[six_p.at[par]], dat_p.at[par], s).wait()
        pltpu.make_async_copy(itab_hbm.at[six_n.at[par]], dat_n.at[par], s).wait()

    def compute(c, par):
        lane = lax.iota(jnp.int32, 16)
        for k in range(_CH // 16):
            uoff = (idx_u[pl.ds(c * _CH + k * 16, 16)] & 1) * 64
            poff = (idx_p[pl.ds(c * _CH + k * 16, 16)] & 1) * 64
            noff = (idx_n[pl.ds(c * _CH + k * 16, 16)] & 1) * 64
            tvec = jnp.zeros((16,), jnp.float32)
            for l in range(16):
                slot = k * 16 + l
                ub, pb, nb = uoff[l], poff[l], noff[l]
                acc = jnp.zeros((16,), jnp.float32)
                for q in range(_D // 16):
                    u = dat_u[par, slot, pl.ds(ub + q * 16, 16)]
                    p = dat_p[par, slot, pl.ds(pb + q * 16, 16)]
                    n = dat_n[par, slot, pl.ds(nb + q * 16, 16)]
                    acc = acc + u * (p - n)
                tvec = jnp.where(lane == l, jnp.sum(acc), tvec)
            scores[pl.ds(c * _CH + k * 16, 16)] = tvec

    fire(0, 0)

    def body(c, carry):
        par = c & 1
        fire(c, par)
        drain(1 - par)
        compute(c - 1, 1 - par)
        return carry

    lax.fori_loop(1, _NCHUNK, body, 0, unroll=False)
    drain((_NCHUNK - 1) & 1)
    compute(_NCHUNK - 1, (_NCHUNK - 1) & 1)

    pltpu.sync_copy(scores, out_hbm.at[pl.ds(base, _RPT)])


def _loss_body(x_ref, o_ref):
    x = x_ref[...]
    z = jnp.exp(-jnp.abs(x))
    ls = jnp.minimum(x, 0.0) - jnp.log(1.0 + z)
    o_ref[0, 0] = -jnp.sum(ls)


def kernel(user_id, pos_id, neg_id, user_table, item_table):
    tail_u = user_table[_V - 64:].reshape(32, 128)
    tail_i = item_table[_V - 64:].reshape(32, 128)
    ut2, it2 = _sc_relayout(user_table.T, item_table.T, tail_u, tail_i)
    tmp = _sc_scores(user_id, pos_id, neg_id, ut2, it2)
    loss = pl.pallas_call(
        _loss_body,
        out_shape=jax.ShapeDtypeStruct((1, 1), jnp.float32),
        out_specs=pl.BlockSpec(memory_space=pltpu.SMEM),
    )(tmp.reshape(128, 128))
    return loss[0, 0]


# speedup vs baseline: 2.4775x; 2.4775x over previous
import functools

import jax
import jax.numpy as jnp
from jax import lax
from jax.experimental import pallas as pl
from jax.experimental.pallas import tpu as pltpu
from jax.experimental.pallas import tpu_sc as plsc

_B = 16384
_D = 64
_V = 1000000
_NC = 2
_NS = 16
_NW = _NC * _NS
_RPT = _B // _NW
_CH = 128
_NCHUNK = _RPT // _CH

_TC_SUB = 512
_TC_COLS = 6144
_TC_GRID = -(-_V // _TC_COLS)

_mesh = plsc.VectorSubcoreMesh(core_axis_name="c", subcore_axis_name="s")


def _tr_body(x_ref, o_ref):
    row = lax.broadcasted_iota(jnp.int32, (_D, _D), 0)
    col = lax.broadcasted_iota(jnp.int32, (_D, _D), 1)
    eye = (row == col).astype(jnp.float32)
    for c in range(_TC_COLS // _TC_SUB):
        x = x_ref[:, pl.ds(c * _TC_SUB, _TC_SUB)]
        o_ref[pl.ds(c * _TC_SUB, _TC_SUB), :] = jax.lax.dot_general(
            x, eye, (((0,), (0,)), ((), ())),
            preferred_element_type=jnp.float32,
        )


def _to_row_major(tab_t):
    return pl.pallas_call(
        _tr_body,
        grid=(_TC_GRID,),
        in_specs=[pl.BlockSpec((_D, _TC_COLS), lambda i: (0, i))],
        out_specs=pl.BlockSpec((_TC_COLS, _D), lambda i: (i, 0)),
        out_shape=jax.ShapeDtypeStruct((_V, _D), jnp.float32),
    )(tab_t)


@functools.partial(
    pl.kernel,
    mesh=_mesh,
    compiler_params=pltpu.CompilerParams(
        needs_layout_passes=False, use_tc_tiling_on_sc=False
    ),
    out_type=jax.ShapeDtypeStruct((_B,), jnp.float32),
    scratch_types=[
        pltpu.VMEM((_NCHUNK, _CH), jnp.int32),
        pltpu.VMEM((_NCHUNK, _CH), jnp.int32),
        pltpu.VMEM((_NCHUNK, _CH), jnp.int32),
        pltpu.VMEM((_RPT, _D), jnp.float32),
        pltpu.VMEM((_RPT, _D), jnp.float32),
        pltpu.VMEM((_RPT, _D), jnp.float32),
        pltpu.VMEM((_RPT,), jnp.float32),
        pltpu.SemaphoreType.DMA,
    ],
)
def _sc_scores(uid_hbm, pid_hbm, nid_hbm, utab_hbm, itab_hbm, out_hbm,
               idx_u, idx_p, idx_n, rows_u, rows_p, rows_n, tmp_v, sem):
    wid = lax.axis_index("s") * _NC + lax.axis_index("c")
    base = wid * _RPT

    for j in range(_NCHUNK):
        off = base + j * _CH
        pltpu.sync_copy(uid_hbm.at[pl.ds(off, _CH)], idx_u.at[j])
        pltpu.sync_copy(pid_hbm.at[pl.ds(off, _CH)], idx_p.at[j])
        pltpu.sync_copy(nid_hbm.at[pl.ds(off, _CH)], idx_n.at[j])

    copies = []
    for j in range(_NCHUNK):
        r = pl.ds(j * _CH, _CH)
        copies.append(pltpu.async_copy(utab_hbm.at[idx_u.at[j]], rows_u.at[r], sem))
        copies.append(pltpu.async_copy(itab_hbm.at[idx_p.at[j]], rows_p.at[r], sem))
        copies.append(pltpu.async_copy(itab_hbm.at[idx_n.at[j]], rows_n.at[r], sem))
    for c in copies:
        c.wait()

    lane = lax.iota(jnp.int32, 16)

    def body(g, carry):
        tvec = jnp.zeros((16,), jnp.float32)
        for l in range(16):
            r = g * 16 + l
            acc = jnp.zeros((16,), jnp.float32)
            for k in range(_D // 16):
                sl = pl.ds(k * 16, 16)
                u = rows_u[r, sl]
                p = rows_p[r, sl]
                n = rows_n[r, sl]
                acc = acc + u * (p - n)
            tvec = jnp.where(lane == l, jnp.sum(acc), tvec)
        tmp_v[pl.ds(g * 16, 16)] = tvec
        return carry

    lax.fori_loop(0, _RPT // 16, body, 0)
    pltpu.sync_copy(tmp_v, out_hbm.at[pl.ds(base, _RPT)])


def _loss_body(x_ref, o_ref):
    x = x_ref[...]
    z = jnp.exp(-jnp.abs(x))
    ls = jnp.minimum(x, 0.0) - jnp.log(1.0 + z)
    o_ref[0, 0] = -jnp.sum(ls)


def kernel(user_id, pos_id, neg_id, user_table, item_table):
    utab = _to_row_major(user_table.T)
    itab = _to_row_major(item_table.T)
    tmp = _sc_scores(user_id, pos_id, neg_id, utab, itab)
    loss = pl.pallas_call(
        _loss_body,
        out_shape=jax.ShapeDtypeStruct((1, 1), jnp.float32),
        out_specs=pl.BlockSpec(memory_space=pltpu.SMEM),
    )(tmp.reshape(128, 128))
    return loss[0, 0]


# speedup vs baseline: 3.0894x; 1.2470x over previous
import functools

import jax
import jax.numpy as jnp
from jax import lax
from jax.experimental import pallas as pl
from jax.experimental.pallas import tpu as pltpu
from jax.experimental.pallas import tpu_sc as plsc

_B = 16384
_D = 64
_NC = 2
_NS = 16
_NW = _NC * _NS
_RPT = _B // _NW
_CH = 128
_NCHUNK = _RPT // _CH

_mesh = plsc.VectorSubcoreMesh(core_axis_name="c", subcore_axis_name="s")


@functools.partial(
    pl.kernel,
    mesh=_mesh,
    compiler_params=pltpu.CompilerParams(
        needs_layout_passes=False, use_tc_tiling_on_sc=False
    ),
    out_type=jax.ShapeDtypeStruct((_B,), jnp.float32),
    scratch_types=[
        pltpu.VMEM((_NCHUNK, _CH), jnp.int32),
        pltpu.VMEM((_NCHUNK, _CH), jnp.int32),
        pltpu.VMEM((_NCHUNK, _CH), jnp.int32),
        pltpu.VMEM((_RPT, _D), jnp.float32),
        pltpu.VMEM((_RPT, _D), jnp.float32),
        pltpu.VMEM((_RPT, _D), jnp.float32),
        pltpu.VMEM((_RPT,), jnp.float32),
        pltpu.SemaphoreType.DMA,
    ],
)
def _sc_scores(uid_hbm, pid_hbm, nid_hbm, utab_hbm, itab_hbm, out_hbm,
               idx_u, idx_p, idx_n, rows_u, rows_p, rows_n, tmp_v, sem):
    wid = lax.axis_index("s") * _NC + lax.axis_index("c")
    base = wid * _RPT

    for j in range(_NCHUNK):
        off = base + j * _CH
        pltpu.sync_copy(uid_hbm.at[pl.ds(off, _CH)], idx_u.at[j])
        pltpu.sync_copy(pid_hbm.at[pl.ds(off, _CH)], idx_p.at[j])
        pltpu.sync_copy(nid_hbm.at[pl.ds(off, _CH)], idx_n.at[j])

    copies = []
    for j in range(_NCHUNK):
        r = pl.ds(j * _CH, _CH)
        copies.append(pltpu.async_copy(utab_hbm.at[idx_u.at[j]], rows_u.at[r], sem))
        copies.append(pltpu.async_copy(itab_hbm.at[idx_p.at[j]], rows_p.at[r], sem))
        copies.append(pltpu.async_copy(itab_hbm.at[idx_n.at[j]], rows_n.at[r], sem))
    for c in copies:
        c.wait()

    lane = lax.iota(jnp.int32, 16)

    def body(g, carry):
        tvec = jnp.zeros((16,), jnp.float32)
        for l in range(16):
            r = g * 16 + l
            acc = jnp.zeros((16,), jnp.float32)
            for k in range(_D // 16):
                sl = pl.ds(k * 16, 16)
                u = rows_u[r, sl]
                p = rows_p[r, sl]
                n = rows_n[r, sl]
                acc = acc + u * (p - n)
            tvec = jnp.where(lane == l, jnp.sum(acc), tvec)
        tmp_v[pl.ds(g * 16, 16)] = tvec
        return carry

    lax.fori_loop(0, _RPT // 16, body, 0)
    pltpu.sync_copy(tmp_v, out_hbm.at[pl.ds(base, _RPT)])


def _loss_body(x_ref, o_ref):
    x = x_ref[...]
    z = jnp.exp(-jnp.abs(x))
    ls = jnp.minimum(x, 0.0) - jnp.log(1.0 + z)
    o_ref[0, 0] = -jnp.sum(ls)


def kernel(user_id, pos_id, neg_id, user_table, item_table):
    tmp = _sc_scores(user_id, pos_id, neg_id, user_table, item_table)
    loss = pl.pallas_call(
        _loss_body,
        out_shape=jax.ShapeDtypeStruct((1, 1), jnp.float32),
        out_specs=pl.BlockSpec(memory_space=pltpu.SMEM),
    )(tmp.reshape(128, 128))
    return loss[0, 0]
